# Initial kernel scaffold; baseline (speedup 1.0000x reference)
import jax
import jax.numpy as jnp
from jax.experimental import pallas as pl


def kernel(tokens, vocab_map):
    raise NotImplementedError("write your pallas kernel here")


# speedup vs baseline: 161.5795x; 161.5795x over previous
import functools

import jax
import jax.numpy as jnp
from jax import lax
from jax.experimental import pallas as pl
from jax.experimental.pallas import tpu as pltpu
from jax.experimental.pallas import tpu_sc as plsc

_BATCH = 4096
_SEQ = 200
_VOCAB = 100000
_N = _BATCH * _SEQ

_NC = 2
_NS = 16
_NW = _NC * _NS
_PER = _N // _NW
_LANES = 16
_VOCAB_PAD = 100096


def _lookup_body(tok_hbm, tab_hbm, out_hbm, tab_v, buf_v):
    wid = lax.axis_index("s") * _NC + lax.axis_index("c")
    base = wid * _PER
    pltpu.sync_copy(tab_hbm, tab_v.at[pl.ds(0, _VOCAB)])
    pltpu.sync_copy(tok_hbm.at[pl.ds(base, _PER)], buf_v)

    def body(i, carry):
        off = i * _LANES
        idx = buf_v[pl.ds(off, _LANES)]
        buf_v[pl.ds(off, _LANES)] = plsc.load_gather(tab_v, [idx])
        return carry

    lax.fori_loop(0, _PER // _LANES, body, 0, unroll=8)
    pltpu.sync_copy(buf_v, out_hbm.at[pl.ds(base, _PER)])


@jax.jit
def kernel(tokens, vocab_map):
    flat = tokens.reshape(_N)
    run = functools.partial(
        pl.kernel,
        mesh=plsc.VectorSubcoreMesh(core_axis_name="c", subcore_axis_name="s"),
        out_type=jax.ShapeDtypeStruct((_N,), jnp.int32),
        scratch_types=[
            pltpu.VMEM((_VOCAB_PAD,), jnp.int32),
            pltpu.VMEM((_PER,), jnp.int32),
        ],
        compiler_params=pltpu.CompilerParams(needs_layout_passes=False),
    )(_lookup_body)
    return run(flat, vocab_map).reshape(_BATCH, _SEQ)


# speedup vs baseline: 187.5954x; 1.1610x over previous
import functools

import jax
import jax.numpy as jnp
from jax import lax
from jax.experimental import pallas as pl
from jax.experimental.pallas import tpu as pltpu
from jax.experimental.pallas import tpu_sc as plsc

_BATCH = 4096
_SEQ = 200
_VOCAB = 100000

_NC = 2
_NS = 16
_NW = _NC * _NS
_ROWS_PER = _BATCH // _NW
_LANES = 16
_VOCAB_PAD = 100096
_FULL_VECS = _SEQ // _LANES
_TAIL_OFF = _SEQ - _LANES


_CHUNK_ROWS = 64
_N_CHUNKS = _ROWS_PER // _CHUNK_ROWS


def _lookup_body(tok_hbm, tab_hbm, out_hbm, tab_v, buf_v):
    wid = lax.axis_index("s") * _NC + lax.axis_index("c")
    row0 = wid * _ROWS_PER
    pltpu.sync_copy(tab_hbm, tab_v.at[pl.ds(0, _VOCAB)])

    def body(r, carry):
        tail_tok = buf_v[r, pl.ds(_TAIL_OFF, _LANES)]
        for j in range(_FULL_VECS):
            idx = buf_v[r, pl.ds(j * _LANES, _LANES)]
            buf_v[r, pl.ds(j * _LANES, _LANES)] = plsc.load_gather(tab_v, [idx])
        buf_v[r, pl.ds(_TAIL_OFF, _LANES)] = plsc.load_gather(tab_v, [tail_tok])
        return carry

    for c in range(_N_CHUNKS):
        base = row0 + c * _CHUNK_ROWS
        pltpu.sync_copy(tok_hbm.at[pl.ds(base, _CHUNK_ROWS)], buf_v)
        lax.fori_loop(0, _CHUNK_ROWS, body, 0)
        pltpu.sync_copy(buf_v, out_hbm.at[pl.ds(base, _CHUNK_ROWS)])


@jax.jit
def kernel(tokens, vocab_map):
    run = functools.partial(
        pl.kernel,
        mesh=plsc.VectorSubcoreMesh(core_axis_name="c", subcore_axis_name="s"),
        out_type=jax.ShapeDtypeStruct((_BATCH, _SEQ), jnp.int32),
        scratch_types=[
            pltpu.VMEM((_VOCAB_PAD,), jnp.int32),
            pltpu.VMEM((_CHUNK_ROWS, _SEQ), jnp.int32),
        ],
        compiler_params=pltpu.CompilerParams(needs_layout_passes=False),
    )(_lookup_body)
    return run(tokens, vocab_map)


# speedup vs baseline: 187.7014x; 1.0006x over previous
import functools

import jax
import jax.numpy as jnp
from jax import lax
from jax.experimental import pallas as pl
from jax.experimental.pallas import tpu as pltpu
from jax.experimental.pallas import tpu_sc as plsc

_BATCH = 4096
_SEQ = 200
_VOCAB = 100000

_NC = 2
_NS = 16
_NW = _NC * _NS
_ROWS_PER = _BATCH // _NW
_LANES = 16
_VOCAB_PAD = 100096
_FULL_VECS = _SEQ // _LANES
_TAIL_OFF = _SEQ - _LANES


_CHUNK_ROWS = 64
_N_CHUNKS = _ROWS_PER // _CHUNK_ROWS


def _lookup_body(tok_hbm, tab_hbm, out_hbm, tab_v, buf_v):
    wid = lax.axis_index("s") * _NC + lax.axis_index("c")
    row0 = wid * _ROWS_PER
    pltpu.sync_copy(tab_hbm, tab_v.at[pl.ds(0, _VOCAB)])

    def body(r, carry):
        tail_tok = buf_v[r, pl.ds(_TAIL_OFF, _LANES)]
        for j in range(_FULL_VECS):
            idx = buf_v[r, pl.ds(j * _LANES, _LANES)]
            buf_v[r, pl.ds(j * _LANES, _LANES)] = plsc.load_gather(tab_v, [idx])
        buf_v[r, pl.ds(_TAIL_OFF, _LANES)] = plsc.load_gather(tab_v, [tail_tok])
        return carry

    for c in range(_N_CHUNKS):
        base = row0 + c * _CHUNK_ROWS
        pltpu.sync_copy(tok_hbm.at[pl.ds(base, _CHUNK_ROWS)], buf_v)
        lax.fori_loop(0, _CHUNK_ROWS, body, 0)
        pltpu.sync_copy(buf_v, out_hbm.at[pl.ds(base, _CHUNK_ROWS)])


@jax.jit
def kernel(tokens, vocab_map):
    run = functools.partial(
        pl.kernel,
        mesh=plsc.VectorSubcoreMesh(core_axis_name="c", subcore_axis_name="s"),
        out_type=jax.ShapeDtypeStruct((_BATCH, _SEQ), jnp.int32),
        scratch_types=[
            pltpu.VMEM((_VOCAB_PAD,), jnp.int32),
            pltpu.VMEM((_CHUNK_ROWS, _SEQ), jnp.int32),
        ],
        compiler_params=pltpu.CompilerParams(
            needs_layout_passes=False, use_tc_tiling_on_sc=True
        ),
    )(_lookup_body)
    return run(tokens, vocab_map)


# speedup vs baseline: 238.6905x; 1.2717x over previous
import functools

import jax
import jax.numpy as jnp
from jax import lax
from jax.experimental import pallas as pl
from jax.experimental.pallas import tpu as pltpu
from jax.experimental.pallas import tpu_sc as plsc

_BATCH = 4096
_SEQ = 200
_VOCAB = 100000

_NC = 2
_NS = 16
_NW = _NC * _NS
_COLS_PER = _BATCH // _NW
_LANES = 16
_VOCAB_PAD = 100096
_VECS_PER_ROW = _COLS_PER // _LANES


def _lookup_body(tok_hbm, tab_hbm, out_hbm, tab_v, buf_v):
    wid = lax.axis_index("s") * _NC + lax.axis_index("c")
    col0 = wid * _COLS_PER
    pltpu.sync_copy(tab_hbm, tab_v.at[pl.ds(0, _VOCAB)])
    pltpu.sync_copy(tok_hbm.at[:, pl.ds(col0, _COLS_PER)], buf_v)

    def body(r, carry):
        for j in range(_VECS_PER_ROW):
            idx = buf_v[r, pl.ds(j * _LANES, _LANES)]
            buf_v[r, pl.ds(j * _LANES, _LANES)] = plsc.load_gather(tab_v, [idx])
        return carry

    lax.fori_loop(0, _SEQ, body, 0)
    pltpu.sync_copy(buf_v, out_hbm.at[:, pl.ds(col0, _COLS_PER)])


@jax.jit
def kernel(tokens, vocab_map):
    run = functools.partial(
        pl.kernel,
        mesh=plsc.VectorSubcoreMesh(core_axis_name="c", subcore_axis_name="s"),
        out_type=jax.ShapeDtypeStruct((_SEQ, _BATCH), jnp.int32),
        scratch_types=[
            pltpu.VMEM((_VOCAB_PAD,), jnp.int32),
            pltpu.VMEM((_SEQ, _COLS_PER), jnp.int32),
        ],
        compiler_params=pltpu.CompilerParams(
            needs_layout_passes=False, use_tc_tiling_on_sc=True
        ),
    )(_lookup_body)
    return run(tokens.T, vocab_map).T


# speedup vs baseline: 249.4178x; 1.0449x over previous
import functools

import jax
import jax.numpy as jnp
from jax import lax
from jax.experimental import pallas as pl
from jax.experimental.pallas import tpu as pltpu
from jax.experimental.pallas import tpu_sc as plsc

_BATCH = 4096
_SEQ = 200
_VOCAB = 100000

_NC = 2
_NS = 16
_NW = _NC * _NS
_COLS_PER = _BATCH // _NW
_LANES = 16
_VOCAB_PAD = 100096
_VECS_PER_ROW = _COLS_PER // _LANES


_CHUNK_ROWS = 40
_N_CHUNKS = _SEQ // _CHUNK_ROWS


def _lookup_body(tok_hbm, tab_hbm, out_hbm, tab_v,
                 i0, i1, o0, o1, sem_tab, si0, si1, so0, so1):
    in_bufs = (i0, i1)
    out_bufs = (o0, o1)
    sin = (si0, si1)
    sout = (so0, so1)
    wid = lax.axis_index("s") * _NC + lax.axis_index("c")
    col0 = wid * _COLS_PER

    tab_dma = pltpu.async_copy(tab_hbm, tab_v.at[pl.ds(0, _VOCAB)], sem_tab)

    def hbm_slice(c):
        return (pl.ds(c * _CHUNK_ROWS, _CHUNK_ROWS), pl.ds(col0, _COLS_PER))

    def start_in(c):
        return pltpu.async_copy(
            tok_hbm.at[hbm_slice(c)], in_bufs[c % 2], sin[c % 2])

    def start_out(c):
        return pltpu.async_copy(
            out_bufs[c % 2], out_hbm.at[hbm_slice(c)], sout[c % 2])

    in_dmas = {c: start_in(c) for c in range(2)}
    out_dmas = {}
    tab_dma.wait()

    def body(src, dst, r, carry):
        for j in range(_VECS_PER_ROW):
            idx = src[r, pl.ds(j * _LANES, _LANES)]
            dst[r, pl.ds(j * _LANES, _LANES)] = plsc.load_gather(tab_v, [idx])
        return carry

    for c in range(_N_CHUNKS):
        in_dmas[c].wait()
        if c - 2 in out_dmas:
            out_dmas[c - 2].wait()
        lax.fori_loop(
            0, _CHUNK_ROWS,
            functools.partial(body, in_bufs[c % 2], out_bufs[c % 2]), 0)
        out_dmas[c] = start_out(c)
        if c + 2 < _N_CHUNKS:
            in_dmas[c + 2] = start_in(c + 2)
    out_dmas[_N_CHUNKS - 2].wait()
    out_dmas[_N_CHUNKS - 1].wait()


@jax.jit
def kernel(tokens, vocab_map):
    run = functools.partial(
        pl.kernel,
        mesh=plsc.VectorSubcoreMesh(core_axis_name="c", subcore_axis_name="s"),
        out_type=jax.ShapeDtypeStruct((_SEQ, _BATCH), jnp.int32),
        scratch_types=[
            pltpu.VMEM((_VOCAB_PAD,), jnp.int32),
            pltpu.VMEM((_CHUNK_ROWS, _COLS_PER), jnp.int32),
            pltpu.VMEM((_CHUNK_ROWS, _COLS_PER), jnp.int32),
            pltpu.VMEM((_CHUNK_ROWS, _COLS_PER), jnp.int32),
            pltpu.VMEM((_CHUNK_ROWS, _COLS_PER), jnp.int32),
            pltpu.SemaphoreType.DMA,
            pltpu.SemaphoreType.DMA,
            pltpu.SemaphoreType.DMA,
            pltpu.SemaphoreType.DMA,
            pltpu.SemaphoreType.DMA,
        ],
        compiler_params=pltpu.CompilerParams(
            needs_layout_passes=False, use_tc_tiling_on_sc=True
        ),
    )(_lookup_body)
    return run(tokens.T, vocab_map).T


# speedup vs baseline: 258.6899x; 1.0372x over previous
import functools

import jax
import jax.numpy as jnp
from jax import lax
from jax.experimental import pallas as pl
from jax.experimental.pallas import tpu as pltpu
from jax.experimental.pallas import tpu_sc as plsc

_BATCH = 4096
_SEQ = 200
_VOCAB = 100000

_NC = 2
_NS = 16
_NW = _NC * _NS
_COLS_PER = _BATCH // _NW
_LANES = 16
_VOCAB_PAD = 100096
_VECS_PER_ROW = _COLS_PER // _LANES


_CHUNK_ROWS = 40
_N_CHUNKS = _SEQ // _CHUNK_ROWS


def _lookup_body(tok_hbm, tab_hbm, out_hbm, tab_v,
                 i0, i1, o0, o1, sem_tab, si0, si1, so0, so1):
    in_bufs = (i0, i1)
    out_bufs = (o0, o1)
    sin = (si0, si1)
    sout = (so0, so1)
    wid = lax.axis_index("s") * _NC + lax.axis_index("c")
    col0 = wid * _COLS_PER

    tab_dma = pltpu.async_copy(tab_hbm, tab_v.at[pl.ds(0, _VOCAB)], sem_tab)

    def hbm_slice(c):
        return (pl.ds(c * _CHUNK_ROWS, _CHUNK_ROWS), pl.ds(col0, _COLS_PER))

    def start_in(c):
        return pltpu.async_copy(
            tok_hbm.at[hbm_slice(c)], in_bufs[c % 2], sin[c % 2])

    def start_out(c):
        return pltpu.async_copy(
            out_bufs[c % 2], out_hbm.at[hbm_slice(c)], sout[c % 2])

    in_dmas = {c: start_in(c) for c in range(2)}
    out_dmas = {}
    tab_dma.wait()

    for c in range(_N_CHUNKS):
        in_dmas[c].wait()
        if c - 2 in out_dmas:
            out_dmas[c - 2].wait()
        src, dst = in_bufs[c % 2], out_bufs[c % 2]

        @plsc.parallel_loop(0, _CHUNK_ROWS, 1, unroll=4)
        def _gather_row(r, src=src, dst=dst):
            for j in range(_VECS_PER_ROW):
                idx = src[r, pl.ds(j * _LANES, _LANES)]
                dst[r, pl.ds(j * _LANES, _LANES)] = plsc.load_gather(tab_v, [idx])

        out_dmas[c] = start_out(c)
        if c + 2 < _N_CHUNKS:
            in_dmas[c + 2] = start_in(c + 2)
    out_dmas[_N_CHUNKS - 2].wait()
    out_dmas[_N_CHUNKS - 1].wait()


@jax.jit
def kernel(tokens, vocab_map):
    run = functools.partial(
        pl.kernel,
        mesh=plsc.VectorSubcoreMesh(core_axis_name="c", subcore_axis_name="s"),
        out_type=jax.ShapeDtypeStruct((_SEQ, _BATCH), jnp.int32),
        scratch_types=[
            pltpu.VMEM((_VOCAB_PAD,), jnp.int32),
            pltpu.VMEM((_CHUNK_ROWS, _COLS_PER), jnp.int32),
            pltpu.VMEM((_CHUNK_ROWS, _COLS_PER), jnp.int32),
            pltpu.VMEM((_CHUNK_ROWS, _COLS_PER), jnp.int32),
            pltpu.VMEM((_CHUNK_ROWS, _COLS_PER), jnp.int32),
            pltpu.SemaphoreType.DMA,
            pltpu.SemaphoreType.DMA,
            pltpu.SemaphoreType.DMA,
            pltpu.SemaphoreType.DMA,
            pltpu.SemaphoreType.DMA,
        ],
        compiler_params=pltpu.CompilerParams(
            needs_layout_passes=False, use_tc_tiling_on_sc=True
        ),
    )(_lookup_body)
    return run(tokens.T, vocab_map).T


# speedup vs baseline: 322.1212x; 1.2452x over previous
import functools

import jax
import jax.numpy as jnp
from jax import lax
from jax.experimental import pallas as pl
from jax.experimental.pallas import tpu as pltpu
from jax.experimental.pallas import tpu_sc as plsc

_BATCH = 4096
_SEQ = 200
_VOCAB = 100000

_NC = 2
_NS = 16
_NW = _NC * _NS
_COLS_PER = _BATCH // _NW
_LANES = 16
_VOCAB_PAD = 100096
_VECS_PER_ROW = _COLS_PER // _LANES


_CHUNK_ROWS = 40
_N_CHUNKS = _SEQ // _CHUNK_ROWS


_TAB_SLICE = _VOCAB_PAD // _NS
_LAST_START = _VOCAB - _TAB_SLICE


def _lookup_body(tok_hbm, tab_hbm, out_hbm, spm_tab, tab_v, slice_v,
                 i0, i1, i2, sem_tab, si0, si1, si2, so0, so1, so2):
    bufs = (i0, i1, i2)
    sin = (si0, si1, si2)
    sout = (so0, so1, so2)
    sid = lax.axis_index("s")
    wid = sid * _NC + lax.axis_index("c")
    col0 = wid * _COLS_PER

    def hbm_slice(c):
        return (pl.ds(c * _CHUNK_ROWS, _CHUNK_ROWS), pl.ds(col0, _COLS_PER))

    def start_in(c):
        return pltpu.async_copy(
            tok_hbm.at[hbm_slice(c)], bufs[c % 3], sin[c % 3])

    def start_out(c):
        return pltpu.async_copy(
            bufs[c % 3], out_hbm.at[hbm_slice(c)], sout[c % 3])

    in_dmas = {c: start_in(c) for c in range(3)}
    out_dmas = {}

    start = pl.multiple_of(
        jnp.where(sid == _NS - 1, _LAST_START, sid * _TAB_SLICE), 8)
    pltpu.async_copy(
        tab_hbm.at[pl.ds(start, _TAB_SLICE)], slice_v, sem_tab).wait()
    pltpu.async_copy(
        slice_v, spm_tab.at[pl.ds(start, _TAB_SLICE)], sem_tab).wait()
    plsc.subcore_barrier()
    pltpu.sync_copy(spm_tab, tab_v)

    for c in range(_N_CHUNKS):
        if 1 <= c and c + 2 < _N_CHUNKS:
            out_dmas[c - 1].wait()
            in_dmas[c + 2] = start_in(c + 2)
        in_dmas[c].wait()
        buf = bufs[c % 3]

        @plsc.parallel_loop(0, _CHUNK_ROWS, 1, unroll=4)
        def _gather_row(r, buf=buf):
            for j in range(_VECS_PER_ROW):
                idx = buf[r, pl.ds(j * _LANES, _LANES)]
                buf[r, pl.ds(j * _LANES, _LANES)] = plsc.load_gather(tab_v, [idx])

        out_dmas[c] = start_out(c)
    for c in range(_N_CHUNKS):
        if c not in (0, 1):
            out_dmas[c].wait()


@jax.jit
def kernel(tokens, vocab_map):
    run = functools.partial(
        pl.kernel,
        mesh=plsc.VectorSubcoreMesh(core_axis_name="c", subcore_axis_name="s"),
        out_type=jax.ShapeDtypeStruct((_SEQ, _BATCH), jnp.int32),
        scratch_types=[
            pltpu.VMEM_SHARED((_VOCAB_PAD,), jnp.int32),
            pltpu.VMEM((_VOCAB_PAD,), jnp.int32),
            pltpu.VMEM((_TAB_SLICE,), jnp.int32),
            pltpu.VMEM((_CHUNK_ROWS, _COLS_PER), jnp.int32),
            pltpu.VMEM((_CHUNK_ROWS, _COLS_PER), jnp.int32),
            pltpu.VMEM((_CHUNK_ROWS, _COLS_PER), jnp.int32),
            pltpu.SemaphoreType.DMA,
            pltpu.SemaphoreType.DMA,
            pltpu.SemaphoreType.DMA,
            pltpu.SemaphoreType.DMA,
            pltpu.SemaphoreType.DMA,
            pltpu.SemaphoreType.DMA,
            pltpu.SemaphoreType.DMA,
        ],
        compiler_params=pltpu.CompilerParams(
            needs_layout_passes=False, use_tc_tiling_on_sc=True
        ),
    )(_lookup_body)
    return run(tokens.T, vocab_map).T
